# baseline (speedup 1.0000x reference)
import functools

import jax
import jax.numpy as jnp
from jax import lax
from jax.experimental import pallas as pl
from jax.experimental.pallas import tpu as pltpu
from jax.experimental.pallas import tpu_sc as plsc

H = 1024
E = 16
N_GROUP = 4
GSZ = E // N_GROUP
TOPK_GROUP = 2
INTER = 512
SI = 1024
SCALE = 2.5

BT = 256
LOG_BT = 8


def _router_body(x_ref, wr_ref, br_ref, o_ref):
    logits = jnp.dot(x_ref[...], wr_ref[...], preferred_element_type=jnp.float32)
    o_ref[...] = jax.nn.sigmoid(logits + br_ref[...])


def _moe_body(meta_ref, xg_ref, wg_ref, wgk_ref, wuk_ref, wdk_ref,
              bg_ref, bu_ref, bd_ref, out_ref, *, cap_tiles):
    e = pl.program_id(0)
    i = pl.program_id(1)
    g = e // GSZ

    @pl.when((e == 0) & (i == 0))
    def _zero():
        out_ref[...] = jnp.zeros_like(out_ref)

    nt = meta_ref[1, g]

    @pl.when(i < nt)
    def _compute():
        xt = xg_ref[...].astype(jnp.bfloat16)
        gg = jnp.dot(xt, wgk_ref[0], preferred_element_type=jnp.float32) + bg_ref[0]
        uu = jnp.dot(xt, wuk_ref[0], preferred_element_type=jnp.float32) + bu_ref[0]
        hh = (gg * jax.nn.sigmoid(gg) * uu).astype(jnp.bfloat16)
        dd = jnp.dot(hh, wdk_ref[0], preferred_element_type=jnp.float32) + bd_ref[0]
        row0 = (meta_ref[0, g] + i) * BT
        w_all = wg_ref[pl.ds(row0, BT), :]
        ids = lax.broadcasted_iota(jnp.int32, (1, 128), 1)
        wcol = jnp.sum(w_all * jnp.where(ids == e, 1.0, 0.0), axis=1, keepdims=True)
        out_ref[pl.ds(row0, BT), :] += dd * wcol


def _shared_body(x_ref, wgs_ref, bgs_ref, wus_ref, bus_ref, wds_ref, bds_ref, o_ref):
    xt = x_ref[...].astype(jnp.bfloat16)
    a = jnp.dot(xt, wgs_ref[...], preferred_element_type=jnp.float32) + bgs_ref[...]
    b = jnp.dot(xt, wus_ref[...], preferred_element_type=jnp.float32) + bus_ref[...]
    h = (a * jax.nn.sigmoid(a) * b).astype(jnp.bfloat16)
    o_ref[...] = jnp.dot(h, wds_ref[...], preferred_element_type=jnp.float32) + bds_ref[...]


def _group_select(cols, ebs):
    sfc = [cols[e] + ebs[e] for e in range(E)]
    gsv = []
    for g in range(N_GROUP):
        c0, c1, c2, c3 = sfc[4 * g:4 * g + 4]
        lo01 = jnp.minimum(c0, c1)
        hi01 = jnp.maximum(c0, c1)
        lo23 = jnp.minimum(c2, c3)
        hi23 = jnp.maximum(c2, c3)
        mn = jnp.minimum(lo01, lo23)
        mn2 = jnp.minimum(jnp.maximum(lo01, lo23), jnp.minimum(hi01, hi23))
        gsv.append(c0 + c1 + c2 + c3 - mn - mn2)
    sel = []
    for g in range(N_GROUP):
        beats = jnp.zeros((16,), jnp.int32)
        for j in range(N_GROUP):
            if j == g:
                continue
            if j < g:
                m = gsv[j] >= gsv[g]
            else:
                m = gsv[j] > gsv[g]
            beats = beats + m.astype(jnp.int32)
        sel.append((beats < TOPK_GROUP).astype(jnp.int32))
    return sel


def _make_route_sc(T, cap):
    NW = 16
    TPW = T // NW
    NCH = TPW // 16
    mesh = plsc.VectorSubcoreMesh(
        core_axis_name="c", subcore_axis_name="s", num_cores=1)

    def body(scores_hbm, x_hbm, ebias_hbm, xg_hbm, wg_hbm, pos_hbm, meta_hbm,
             sc_scores, xbuf, cmbbuf, pb0, pb1, ebv, metas):
        wid = lax.axis_index("s")
        base = wid * TPW
        iota16 = lax.iota(jnp.int32, 16)
        onehot = [(iota16 == g).astype(jnp.int32) for g in range(N_GROUP)]
        pltpu.sync_copy(ebias_hbm, ebv)
        ebvec = ebv[...]
        ebs = [jnp.take_along_axis(ebvec, jnp.full((16,), e, jnp.int32), axis=0)
               for e in range(E)]

        def get_cols(ch):
            rows = ch * 16 + iota16
            return [plsc.load_gather(
                sc_scores, [rows, jnp.full((16,), e, jnp.int32)])
                for e in range(E)]

        def blk_body(blk, carry):
            pre, tot = carry
            pltpu.sync_copy(scores_hbm.at[pl.ds(blk * TPW, TPW)], sc_scores)
            cntb = jnp.zeros((16,), jnp.int32)
            for ch in range(NCH):
                sel = _group_select(get_cols(ch), ebs)
                for g in range(N_GROUP):
                    cntb = cntb + jnp.sum(sel[g]) * onehot[g]
            tot = tot + cntb
            pre = pre + jnp.where(blk < wid, cntb, 0)
            return pre, tot

        pre, tot = lax.fori_loop(
            0, NW, blk_body,
            (jnp.zeros((16,), jnp.int32), jnp.zeros((16,), jnp.int32)))
        pltpu.sync_copy(scores_hbm.at[pl.ds(base, TPW)], sc_scores)
        aligned = ((tot + BT - 1) >> LOG_BT) << LOG_BT
        aligned = jnp.where(iota16 < N_GROUP, aligned, 0)
        cums = plsc.cumsum(aligned)
        off = cums - aligned
        basev = off + pre

        @pl.when(wid == 0)
        def _meta():
            metas[0, :] = off >> LOG_BT
            metas[1, :] = aligned >> LOG_BT
            pltpu.sync_copy(metas, meta_hbm)

        z16 = jnp.zeros((16,), jnp.float32)
        for r in range(16):
            for k in range(8):
                cmbbuf[r, pl.ds(k * 16, 16)] = z16

        rb = basev
        for ch in range(NCH):
            tok0 = base + ch * 16
            pltpu.sync_copy(x_hbm.at[pl.ds(tok0, 16)], xbuf)
            cols = get_cols(ch)
            sel = _group_select(cols, ebs)
            wts = [cols[e] * sel[e // GSZ].astype(jnp.float32) for e in range(E)]
            denom = functools.reduce(lambda a, b: a + b, wts) + 1e-20
            inv = SCALE / denom
            for e in range(E):
                plsc.store_scatter(
                    cmbbuf, [iota16, jnp.full((16,), e, jnp.int32)], wts[e] * inv)
            before = jnp.zeros((16,), jnp.int32)
            pos0 = jnp.zeros((16,), jnp.int32)
            pos1 = jnp.zeros((16,), jnp.int32)
            for g in range(N_GROUP):
                sg = sel[g]
                excl = plsc.cumsum(sg) - sg
                pg = jnp.take_along_axis(
                    rb, jnp.full((16,), g, jnp.int32), axis=0) + excl
                pos0 = pos0 + jnp.where((sg > 0) & (before == 0), pg, 0)
                pos1 = pos1 + jnp.where((sg > 0) & (before == 1), pg, 0)
                before = before + sg
                rb = rb + jnp.sum(sg) * onehot[g]
            pltpu.sync_copy(xbuf, xg_hbm.at[pos0])
            pltpu.sync_copy(xbuf, xg_hbm.at[pos1])
            pltpu.sync_copy(cmbbuf, wg_hbm.at[pos0])
            pltpu.sync_copy(cmbbuf, wg_hbm.at[pos1])
            pb0[pl.ds(ch * 16, 16)] = pos0
            pb1[pl.ds(ch * 16, 16)] = pos1
        pltpu.sync_copy(pb0, pos_hbm.at[pl.ds(base, TPW)])
        pltpu.sync_copy(pb1, pos_hbm.at[pl.ds(T + base, TPW)])

    return functools.partial(
        pl.kernel, body,
        out_type=[
            jax.ShapeDtypeStruct((cap, H), jnp.float32),
            jax.ShapeDtypeStruct((cap, 128), jnp.float32),
            jax.ShapeDtypeStruct((2 * T,), jnp.int32),
            jax.ShapeDtypeStruct((2, 16), jnp.int32),
        ],
        mesh=mesh,
        scratch_types=[
            pltpu.VMEM((TPW, 128), jnp.float32),
            pltpu.VMEM((16, H), jnp.float32),
            pltpu.VMEM((16, 128), jnp.float32),
            pltpu.VMEM((TPW,), jnp.int32),
            pltpu.VMEM((TPW,), jnp.int32),
            pltpu.VMEM((16,), jnp.float32),
            pltpu.VMEM((2, 16), jnp.int32),
        ],
        compiler_params=pltpu.CompilerParams(needs_layout_passes=False),
    )()


def _make_combine_sc(T, cap):
    NW = 32
    TPW = T // NW
    NCH = TPW // 16
    mesh = plsc.VectorSubcoreMesh(core_axis_name="c", subcore_axis_name="s")

    def body(rs_hbm, shared_hbm, pos_hbm, out_hbm, r0, r1, sh, idx0, idx1, s0, s1):
        cid = lax.axis_index("c")
        sid = lax.axis_index("s")
        wid = sid * 2 + cid
        base = wid * TPW
        for ch in range(NCH):
            t0 = base + ch * 16
            pltpu.sync_copy(pos_hbm.at[pl.ds(t0, 16)], idx0)
            pltpu.sync_copy(pos_hbm.at[pl.ds(T + t0, 16)], idx1)
            c0 = pltpu.async_copy(rs_hbm.at[idx0[...]], r0, s0)
            c1 = pltpu.async_copy(rs_hbm.at[idx1[...]], r1, s1)
            pltpu.sync_copy(shared_hbm.at[pl.ds(t0, 16)], sh)
            c0.wait()
            c1.wait()
            for j in range(16):
                def addrow(k, _, j=j):
                    sl = pl.ds(k * 16, 16)
                    sh[j, sl] = sh[j, sl] + r0[j, sl] + r1[j, sl]
                    return 0
                lax.fori_loop(0, H // 16, addrow, 0)
            pltpu.sync_copy(sh, out_hbm.at[pl.ds(t0, 16)])

    return functools.partial(
        pl.kernel, body,
        out_type=jax.ShapeDtypeStruct((T, H), jnp.float32),
        mesh=mesh,
        scratch_types=[
            pltpu.VMEM((16, H), jnp.float32),
            pltpu.VMEM((16, H), jnp.float32),
            pltpu.VMEM((16, H), jnp.float32),
            pltpu.VMEM((16,), jnp.int32),
            pltpu.VMEM((16,), jnp.int32),
            pltpu.SemaphoreType.DMA,
            pltpu.SemaphoreType.DMA,
        ],
        compiler_params=pltpu.CompilerParams(needs_layout_passes=False),
    )()


def kernel(hidden_states, Wr, br, e_bias, Wg, bg, Wu, bu, Wd, bd, Wgs, bgs, Wus, bus, Wds, bds):
    orig_shape = hidden_states.shape
    x = hidden_states.reshape(-1, H).astype(jnp.float32)
    T = x.shape[0]
    cap = 2 * T + N_GROUP * BT
    cap_tiles = cap // BT
    maxt = T // BT

    wrp = jnp.pad(Wr, ((0, 0), (0, 128 - E)))
    brp = jnp.pad(br, (0, 128 - E)).reshape(1, 128)
    scores = pl.pallas_call(
        _router_body,
        grid=(T // 512,),
        in_specs=[
            pl.BlockSpec((512, H), lambda i: (i, 0)),
            pl.BlockSpec((H, 128), lambda i: (0, 0)),
            pl.BlockSpec((1, 128), lambda i: (0, 0)),
        ],
        out_specs=pl.BlockSpec((512, 128), lambda i: (i, 0)),
        out_shape=jax.ShapeDtypeStruct((T, 128), jnp.float32),
    )(x, wrp, brp)

    xg, wg, pos, meta = _make_route_sc(T, cap)(scores, x, e_bias)

    grid_spec = pltpu.PrefetchScalarGridSpec(
        num_scalar_prefetch=1,
        grid=(E, maxt),
        in_specs=[
            pl.BlockSpec(
                (BT, H),
                lambda e, i, m: (jnp.minimum(m[0, e // GSZ] + i, cap_tiles - 1), 0)),
            pl.BlockSpec((cap, 128), lambda e, i, m: (0, 0)),
            pl.BlockSpec((1, H, INTER), lambda e, i, m: (e, 0, 0)),
            pl.BlockSpec((1, H, INTER), lambda e, i, m: (e, 0, 0)),
            pl.BlockSpec((1, INTER, H), lambda e, i, m: (e, 0, 0)),
            pl.BlockSpec((1, 1, INTER), lambda e, i, m: (e, 0, 0)),
            pl.BlockSpec((1, 1, INTER), lambda e, i, m: (e, 0, 0)),
            pl.BlockSpec((1, 1, H), lambda e, i, m: (e, 0, 0)),
        ],
        out_specs=pl.BlockSpec((cap, H), lambda e, i, m: (0, 0)),
    )
    routed_sorted = pl.pallas_call(
        functools.partial(_moe_body, cap_tiles=cap_tiles),
        grid_spec=grid_spec,
        out_shape=jax.ShapeDtypeStruct((cap, H), jnp.float32),
    )(meta, xg, wg,
      Wg.astype(jnp.bfloat16), Wu.astype(jnp.bfloat16), Wd.astype(jnp.bfloat16),
      bg.reshape(E, 1, INTER), bu.reshape(E, 1, INTER), bd.reshape(E, 1, H))

    shared = pl.pallas_call(
        _shared_body,
        grid=(T // BT,),
        in_specs=[
            pl.BlockSpec((BT, H), lambda i: (i, 0)),
            pl.BlockSpec((H, SI), lambda i: (0, 0)),
            pl.BlockSpec((1, SI), lambda i: (0, 0)),
            pl.BlockSpec((H, SI), lambda i: (0, 0)),
            pl.BlockSpec((1, SI), lambda i: (0, 0)),
            pl.BlockSpec((SI, H), lambda i: (0, 0)),
            pl.BlockSpec((1, H), lambda i: (0, 0)),
        ],
        out_specs=pl.BlockSpec((BT, H), lambda i: (i, 0)),
        out_shape=jax.ShapeDtypeStruct((T, H), jnp.float32),
    )(x, Wgs.astype(jnp.bfloat16), bgs.reshape(1, SI),
      Wus.astype(jnp.bfloat16), bus.reshape(1, SI),
      Wds.astype(jnp.bfloat16), bds.reshape(1, H))

    out = _make_combine_sc(T, cap)(routed_sorted, shared, pos)
    return out.reshape(orig_shape)


# speedup vs baseline: 1.0761x; 1.0761x over previous
import functools

import jax
import jax.numpy as jnp
from jax import lax
from jax.experimental import pallas as pl
from jax.experimental.pallas import tpu as pltpu
from jax.experimental.pallas import tpu_sc as plsc

H = 1024
E = 16
N_GROUP = 4
GSZ = E // N_GROUP
TOPK_GROUP = 2
INTER = 512
SI = 1024
SCALE = 2.5

BT = 256
LOG_BT = 8


def _router_body(x_ref, wr_ref, br_ref, o_ref):
    logits = jnp.dot(x_ref[...], wr_ref[...], preferred_element_type=jnp.float32)
    o_ref[...] = jax.nn.sigmoid(logits + br_ref[...])


def _moe_body(meta_ref, xg_ref, wg_ref, wgk_ref, wuk_ref, wdk_ref,
              bg_ref, bu_ref, bd_ref, out_ref, *, cap_tiles):
    e = pl.program_id(0)
    i = pl.program_id(1)
    g = e // GSZ

    @pl.when((e == 0) & (i == 0))
    def _zero():
        out_ref[...] = jnp.zeros_like(out_ref)

    nt = meta_ref[1, g]

    @pl.when(i < nt)
    def _compute():
        xt = xg_ref[...]
        gg = jnp.dot(xt, wgk_ref[0], preferred_element_type=jnp.float32) + bg_ref[0]
        uu = jnp.dot(xt, wuk_ref[0], preferred_element_type=jnp.float32) + bu_ref[0]
        hh = gg * jax.nn.sigmoid(gg) * uu
        dd = jnp.dot(hh, wdk_ref[0], preferred_element_type=jnp.float32) + bd_ref[0]
        row0 = (meta_ref[0, g] + i) * BT
        w_all = wg_ref[pl.ds(row0, BT), :]
        ids = lax.broadcasted_iota(jnp.int32, (1, 128), 1)
        wcol = jnp.sum(w_all * jnp.where(ids == e, 1.0, 0.0), axis=1, keepdims=True)
        out_ref[pl.ds(row0, BT), :] += dd * wcol


def _shared_body(x_ref, wgs_ref, bgs_ref, wus_ref, bus_ref, wds_ref, bds_ref, o_ref):
    xt = x_ref[...]
    a = jnp.dot(xt, wgs_ref[...], preferred_element_type=jnp.float32) + bgs_ref[...]
    b = jnp.dot(xt, wus_ref[...], preferred_element_type=jnp.float32) + bus_ref[...]
    h = a * jax.nn.sigmoid(a) * b
    o_ref[...] = jnp.dot(h, wds_ref[...], preferred_element_type=jnp.float32) + bds_ref[...]


def _group_select(cols, ebs):
    sfc = [cols[e] + ebs[e] for e in range(E)]
    gsv = []
    for g in range(N_GROUP):
        c0, c1, c2, c3 = sfc[4 * g:4 * g + 4]
        lo01 = jnp.minimum(c0, c1)
        hi01 = jnp.maximum(c0, c1)
        lo23 = jnp.minimum(c2, c3)
        hi23 = jnp.maximum(c2, c3)
        mn = jnp.minimum(lo01, lo23)
        mn2 = jnp.minimum(jnp.maximum(lo01, lo23), jnp.minimum(hi01, hi23))
        gsv.append(c0 + c1 + c2 + c3 - mn - mn2)
    sel = []
    for g in range(N_GROUP):
        beats = jnp.zeros((16,), jnp.int32)
        for j in range(N_GROUP):
            if j == g:
                continue
            if j < g:
                m = gsv[j] >= gsv[g]
            else:
                m = gsv[j] > gsv[g]
            beats = beats + m.astype(jnp.int32)
        sel.append((beats < TOPK_GROUP).astype(jnp.int32))
    return sel


def _make_route_sc(T, cap):
    NW = 32
    TPW = T // NW
    NCH = TPW // 16
    mesh = plsc.VectorSubcoreMesh(core_axis_name="c", subcore_axis_name="s")

    def body(scores_hbm, x_hbm, ebias_hbm, xg_hbm, wg_hbm, pos_hbm, meta_hbm,
             sc_scores, xbuf, cmbbuf, pb0, pb1, ebv, metas):
        wid = lax.axis_index("s") * 2 + lax.axis_index("c")
        base = wid * TPW
        iota16 = lax.iota(jnp.int32, 16)
        onehot = [(iota16 == g).astype(jnp.int32) for g in range(N_GROUP)]
        pltpu.sync_copy(ebias_hbm, ebv)
        ebvec = ebv[...]
        ebs = [jnp.take_along_axis(ebvec, jnp.full((16,), e, jnp.int32), axis=0)
               for e in range(E)]

        def get_cols(ch):
            rows = ch * 16 + iota16
            return [plsc.load_gather(
                sc_scores, [rows, jnp.full((16,), e, jnp.int32)])
                for e in range(E)]

        def blk_body(blk, carry):
            pre, tot = carry
            pltpu.sync_copy(scores_hbm.at[pl.ds(blk * TPW, TPW)], sc_scores)
            cntb = jnp.zeros((16,), jnp.int32)
            for ch in range(NCH):
                sel = _group_select(get_cols(ch), ebs)
                for g in range(N_GROUP):
                    cntb = cntb + jnp.sum(sel[g]) * onehot[g]
            tot = tot + cntb
            pre = pre + jnp.where(blk < wid, cntb, 0)
            return pre, tot

        pre, tot = lax.fori_loop(
            0, NW, blk_body,
            (jnp.zeros((16,), jnp.int32), jnp.zeros((16,), jnp.int32)))
        pltpu.sync_copy(scores_hbm.at[pl.ds(base, TPW)], sc_scores)
        aligned = ((tot + BT - 1) >> LOG_BT) << LOG_BT
        aligned = jnp.where(iota16 < N_GROUP, aligned, 0)
        cums = plsc.cumsum(aligned)
        off = cums - aligned
        basev = off + pre

        @pl.when(wid == 0)
        def _meta():
            metas[0, :] = off >> LOG_BT
            metas[1, :] = aligned >> LOG_BT
            pltpu.sync_copy(metas, meta_hbm)

        z16 = jnp.zeros((16,), jnp.float32)
        for r in range(16):
            for k in range(8):
                cmbbuf[r, pl.ds(k * 16, 16)] = z16

        rb = basev
        for ch in range(NCH):
            tok0 = base + ch * 16
            pltpu.sync_copy(x_hbm.at[pl.ds(tok0, 16)], xbuf)
            cols = get_cols(ch)
            sel = _group_select(cols, ebs)
            wts = [cols[e] * sel[e // GSZ].astype(jnp.float32) for e in range(E)]
            denom = functools.reduce(lambda a, b: a + b, wts) + 1e-20
            inv = SCALE / denom
            for e in range(E):
                plsc.store_scatter(
                    cmbbuf, [iota16, jnp.full((16,), e, jnp.int32)], wts[e] * inv)
            before = jnp.zeros((16,), jnp.int32)
            pos0 = jnp.zeros((16,), jnp.int32)
            pos1 = jnp.zeros((16,), jnp.int32)
            for g in range(N_GROUP):
                sg = sel[g]
                excl = plsc.cumsum(sg) - sg
                pg = jnp.take_along_axis(
                    rb, jnp.full((16,), g, jnp.int32), axis=0) + excl
                pos0 = pos0 + jnp.where((sg > 0) & (before == 0), pg, 0)
                pos1 = pos1 + jnp.where((sg > 0) & (before == 1), pg, 0)
                before = before + sg
                rb = rb + jnp.sum(sg) * onehot[g]
            pltpu.sync_copy(xbuf, xg_hbm.at[pos0])
            pltpu.sync_copy(xbuf, xg_hbm.at[pos1])
            pltpu.sync_copy(cmbbuf, wg_hbm.at[pos0])
            pltpu.sync_copy(cmbbuf, wg_hbm.at[pos1])
            pb0[pl.ds(ch * 16, 16)] = pos0
            pb1[pl.ds(ch * 16, 16)] = pos1
        pltpu.sync_copy(pb0, pos_hbm.at[pl.ds(base, TPW)])
        pltpu.sync_copy(pb1, pos_hbm.at[pl.ds(T + base, TPW)])

    return functools.partial(
        pl.kernel, body,
        out_type=[
            jax.ShapeDtypeStruct((cap, H), jnp.float32),
            jax.ShapeDtypeStruct((cap, 128), jnp.float32),
            jax.ShapeDtypeStruct((2 * T,), jnp.int32),
            jax.ShapeDtypeStruct((2, 16), jnp.int32),
        ],
        mesh=mesh,
        scratch_types=[
            pltpu.VMEM((TPW, 128), jnp.float32),
            pltpu.VMEM((16, H), jnp.float32),
            pltpu.VMEM((16, 128), jnp.float32),
            pltpu.VMEM((TPW,), jnp.int32),
            pltpu.VMEM((TPW,), jnp.int32),
            pltpu.VMEM((16,), jnp.float32),
            pltpu.VMEM((2, 16), jnp.int32),
        ],
        compiler_params=pltpu.CompilerParams(needs_layout_passes=False),
    )()


def _make_combine_sc(T, cap):
    NW = 32
    TPW = T // NW
    NCH = TPW // 16
    mesh = plsc.VectorSubcoreMesh(core_axis_name="c", subcore_axis_name="s")

    def body(rs_hbm, shared_hbm, pos_hbm, out_hbm, r0, r1, sh, idx0, idx1, s0, s1):
        cid = lax.axis_index("c")
        sid = lax.axis_index("s")
        wid = sid * 2 + cid
        base = wid * TPW
        for ch in range(NCH):
            t0 = base + ch * 16
            pltpu.sync_copy(pos_hbm.at[pl.ds(t0, 16)], idx0)
            pltpu.sync_copy(pos_hbm.at[pl.ds(T + t0, 16)], idx1)
            c0 = pltpu.async_copy(rs_hbm.at[idx0[...]], r0, s0)
            c1 = pltpu.async_copy(rs_hbm.at[idx1[...]], r1, s1)
            pltpu.sync_copy(shared_hbm.at[pl.ds(t0, 16)], sh)
            c0.wait()
            c1.wait()
            for j in range(16):
                def addrow(k, _, j=j):
                    sl = pl.ds(k * 16, 16)
                    sh[j, sl] = sh[j, sl] + r0[j, sl] + r1[j, sl]
                    return 0
                lax.fori_loop(0, H // 16, addrow, 0)
            pltpu.sync_copy(sh, out_hbm.at[pl.ds(t0, 16)])

    return functools.partial(
        pl.kernel, body,
        out_type=jax.ShapeDtypeStruct((T, H), jnp.float32),
        mesh=mesh,
        scratch_types=[
            pltpu.VMEM((16, H), jnp.float32),
            pltpu.VMEM((16, H), jnp.float32),
            pltpu.VMEM((16, H), jnp.float32),
            pltpu.VMEM((16,), jnp.int32),
            pltpu.VMEM((16,), jnp.int32),
            pltpu.SemaphoreType.DMA,
            pltpu.SemaphoreType.DMA,
        ],
        compiler_params=pltpu.CompilerParams(needs_layout_passes=False),
    )()


def kernel(hidden_states, Wr, br, e_bias, Wg, bg, Wu, bu, Wd, bd, Wgs, bgs, Wus, bus, Wds, bds):
    orig_shape = hidden_states.shape
    x = hidden_states.reshape(-1, H).astype(jnp.float32)
    T = x.shape[0]
    cap = 2 * T + N_GROUP * BT
    cap_tiles = cap // BT
    maxt = T // BT

    wrp = jnp.pad(Wr, ((0, 0), (0, 128 - E)))
    brp = jnp.pad(br, (0, 128 - E)).reshape(1, 128)
    scores = pl.pallas_call(
        _router_body,
        grid=(T // 512,),
        in_specs=[
            pl.BlockSpec((512, H), lambda i: (i, 0)),
            pl.BlockSpec((H, 128), lambda i: (0, 0)),
            pl.BlockSpec((1, 128), lambda i: (0, 0)),
        ],
        out_specs=pl.BlockSpec((512, 128), lambda i: (i, 0)),
        out_shape=jax.ShapeDtypeStruct((T, 128), jnp.float32),
    )(x, wrp, brp)

    xg, wg, pos, meta = _make_route_sc(T, cap)(scores, x, e_bias)

    grid_spec = pltpu.PrefetchScalarGridSpec(
        num_scalar_prefetch=1,
        grid=(E, maxt),
        in_specs=[
            pl.BlockSpec(
                (BT, H),
                lambda e, i, m: (m[0, e // GSZ] + jnp.minimum(i, jnp.maximum(m[1, e // GSZ] - 1, 0)), 0)),
            pl.BlockSpec((cap, 128), lambda e, i, m: (0, 0)),
            pl.BlockSpec((1, H, INTER), lambda e, i, m: (e, 0, 0)),
            pl.BlockSpec((1, H, INTER), lambda e, i, m: (e, 0, 0)),
            pl.BlockSpec((1, INTER, H), lambda e, i, m: (e, 0, 0)),
            pl.BlockSpec((1, 1, INTER), lambda e, i, m: (e, 0, 0)),
            pl.BlockSpec((1, 1, INTER), lambda e, i, m: (e, 0, 0)),
            pl.BlockSpec((1, 1, H), lambda e, i, m: (e, 0, 0)),
        ],
        out_specs=pl.BlockSpec((cap, H), lambda e, i, m: (0, 0)),
    )
    routed_sorted = pl.pallas_call(
        functools.partial(_moe_body, cap_tiles=cap_tiles),
        grid_spec=grid_spec,
        out_shape=jax.ShapeDtypeStruct((cap, H), jnp.float32),
    )(meta, xg, wg, Wg, Wu, Wd,
      bg.reshape(E, 1, INTER), bu.reshape(E, 1, INTER), bd.reshape(E, 1, H))

    shared = pl.pallas_call(
        _shared_body,
        grid=(T // BT,),
        in_specs=[
            pl.BlockSpec((BT, H), lambda i: (i, 0)),
            pl.BlockSpec((H, SI), lambda i: (0, 0)),
            pl.BlockSpec((1, SI), lambda i: (0, 0)),
            pl.BlockSpec((H, SI), lambda i: (0, 0)),
            pl.BlockSpec((1, SI), lambda i: (0, 0)),
            pl.BlockSpec((SI, H), lambda i: (0, 0)),
            pl.BlockSpec((1, H), lambda i: (0, 0)),
        ],
        out_specs=pl.BlockSpec((BT, H), lambda i: (i, 0)),
        out_shape=jax.ShapeDtypeStruct((T, H), jnp.float32),
    )(x, Wgs, bgs.reshape(1, SI), Wus, bus.reshape(1, SI), Wds, bds.reshape(1, H))

    out = _make_combine_sc(T, cap)(routed_sorted, shared, pos)
    return out.reshape(orig_shape)


# speedup vs baseline: 1.1218x; 1.0425x over previous
import functools

import jax
import jax.numpy as jnp
from jax import lax
from jax.experimental import pallas as pl
from jax.experimental.pallas import tpu as pltpu
from jax.experimental.pallas import tpu_sc as plsc

H = 1024
E = 16
N_GROUP = 4
GSZ = E // N_GROUP
TOPK_GROUP = 2
INTER = 512
SI = 1024
SCALE = 2.5

BT = 256
LOG_BT = 8


def _router_body(x_ref, wr_ref, br_ref, o_ref):
    logits = jnp.dot(x_ref[...], wr_ref[...], preferred_element_type=jnp.float32)
    o_ref[...] = jax.nn.sigmoid(logits + br_ref[...])


def _moe_body(meta_ref, xg_ref, wg_ref, wgk_ref, wuk_ref, wdk_ref,
              bg_ref, bu_ref, bd_ref, out_ref, *, cap_tiles):
    e = pl.program_id(0)
    i = pl.program_id(1)
    g = e // GSZ

    @pl.when((e == 0) & (i == 0))
    def _zero():
        out_ref[...] = jnp.zeros_like(out_ref)

    nt = meta_ref[1, g]

    @pl.when(i < nt)
    def _compute():
        xt = xg_ref[...]
        gg = jnp.dot(xt, wgk_ref[0], preferred_element_type=jnp.float32) + bg_ref[0]
        uu = jnp.dot(xt, wuk_ref[0], preferred_element_type=jnp.float32) + bu_ref[0]
        hh = gg * jax.nn.sigmoid(gg) * uu
        dd = jnp.dot(hh, wdk_ref[0], preferred_element_type=jnp.float32) + bd_ref[0]
        row0 = (meta_ref[0, g] + i) * BT
        w_all = wg_ref[pl.ds(row0, BT), :]
        ids = lax.broadcasted_iota(jnp.int32, (1, 128), 1)
        wcol = jnp.sum(w_all * jnp.where(ids == e, 1.0, 0.0), axis=1, keepdims=True)
        out_ref[pl.ds(row0, BT), :] += dd * wcol


def _shared_body(x_ref, wgs_ref, bgs_ref, wus_ref, bus_ref, wds_ref, bds_ref, o_ref):
    xt = x_ref[...]
    a = jnp.dot(xt, wgs_ref[...], preferred_element_type=jnp.float32) + bgs_ref[...]
    b = jnp.dot(xt, wus_ref[...], preferred_element_type=jnp.float32) + bus_ref[...]
    h = a * jax.nn.sigmoid(a) * b
    o_ref[...] = jnp.dot(h, wds_ref[...], preferred_element_type=jnp.float32) + bds_ref[...]


def _group_select(cols, ebs):
    sfc = [cols[e] + ebs[e] for e in range(E)]
    gsv = []
    for g in range(N_GROUP):
        c0, c1, c2, c3 = sfc[4 * g:4 * g + 4]
        lo01 = jnp.minimum(c0, c1)
        hi01 = jnp.maximum(c0, c1)
        lo23 = jnp.minimum(c2, c3)
        hi23 = jnp.maximum(c2, c3)
        mn = jnp.minimum(lo01, lo23)
        mn2 = jnp.minimum(jnp.maximum(lo01, lo23), jnp.minimum(hi01, hi23))
        gsv.append(c0 + c1 + c2 + c3 - mn - mn2)
    sel = []
    for g in range(N_GROUP):
        beats = jnp.zeros((16,), jnp.int32)
        for j in range(N_GROUP):
            if j == g:
                continue
            if j < g:
                m = gsv[j] >= gsv[g]
            else:
                m = gsv[j] > gsv[g]
            beats = beats + m.astype(jnp.int32)
        sel.append((beats < TOPK_GROUP).astype(jnp.int32))
    return sel


def _make_route_sc(T, cap):
    NW = 32
    TPW = T // NW
    NCH = TPW // 16
    mesh = plsc.VectorSubcoreMesh(core_axis_name="c", subcore_axis_name="s")

    def body(scores_hbm, x_hbm, ebias_hbm, xg_hbm, wg_hbm, pos_hbm, meta_hbm,
             sc_scores, xbuf, cmbbuf, pb0, pb1, ebv, metas):
        wid = lax.axis_index("s") * 2 + lax.axis_index("c")
        base = wid * TPW
        iota16 = lax.iota(jnp.int32, 16)
        onehot = [(iota16 == g).astype(jnp.int32) for g in range(N_GROUP)]
        pltpu.sync_copy(ebias_hbm, ebv)
        ebvec = ebv[...]
        ebs = [jnp.take_along_axis(ebvec, jnp.full((16,), e, jnp.int32), axis=0)
               for e in range(E)]

        def get_cols(ch):
            rows = ch * 16 + iota16
            return [plsc.load_gather(
                sc_scores, [rows, jnp.full((16,), e, jnp.int32)])
                for e in range(E)]

        def blk_body(blk, carry):
            pre, tot = carry
            pltpu.sync_copy(scores_hbm.at[pl.ds(blk * TPW, TPW)], sc_scores)
            cntb = jnp.zeros((16,), jnp.int32)
            for ch in range(NCH):
                sel = _group_select(get_cols(ch), ebs)
                for g in range(N_GROUP):
                    cntb = cntb + jnp.sum(sel[g]) * onehot[g]
            tot = tot + cntb
            pre = pre + jnp.where(blk < wid, cntb, 0)
            return pre, tot

        pre, tot = lax.fori_loop(
            0, NW, blk_body,
            (jnp.zeros((16,), jnp.int32), jnp.zeros((16,), jnp.int32)))
        pltpu.sync_copy(scores_hbm.at[pl.ds(base, TPW)], sc_scores)
        aligned = ((tot + BT - 1) >> LOG_BT) << LOG_BT
        aligned = jnp.where(iota16 < N_GROUP, aligned, 0)
        cums = plsc.cumsum(aligned)
        off = cums - aligned
        basev = off + pre

        @pl.when(wid == 0)
        def _meta():
            metas[0, :] = off >> LOG_BT
            metas[1, :] = aligned >> LOG_BT
            pltpu.sync_copy(metas, meta_hbm)

        z16 = jnp.zeros((16,), jnp.float32)
        for r in range(16):
            for k in range(8):
                cmbbuf[r, pl.ds(k * 16, 16)] = z16

        rb = basev
        for ch in range(NCH):
            tok0 = base + ch * 16
            pltpu.sync_copy(x_hbm.at[pl.ds(tok0, 16)], xbuf)
            cols = get_cols(ch)
            sel = _group_select(cols, ebs)
            wts = [cols[e] * sel[e // GSZ].astype(jnp.float32) for e in range(E)]
            denom = functools.reduce(lambda a, b: a + b, wts) + 1e-20
            inv = SCALE / denom
            for e in range(E):
                plsc.store_scatter(
                    cmbbuf, [iota16, jnp.full((16,), e, jnp.int32)], wts[e] * inv)
            before = jnp.zeros((16,), jnp.int32)
            pos0 = jnp.zeros((16,), jnp.int32)
            pos1 = jnp.zeros((16,), jnp.int32)
            for g in range(N_GROUP):
                sg = sel[g]
                excl = plsc.cumsum(sg) - sg
                pg = jnp.take_along_axis(
                    rb, jnp.full((16,), g, jnp.int32), axis=0) + excl
                pos0 = pos0 + jnp.where((sg > 0) & (before == 0), pg, 0)
                pos1 = pos1 + jnp.where((sg > 0) & (before == 1), pg, 0)
                before = before + sg
                rb = rb + jnp.sum(sg) * onehot[g]
            pltpu.sync_copy(xbuf, xg_hbm.at[pos0])
            pltpu.sync_copy(xbuf, xg_hbm.at[pos1])
            pltpu.sync_copy(cmbbuf, wg_hbm.at[pos0])
            pltpu.sync_copy(cmbbuf, wg_hbm.at[pos1])
            pb0[pl.ds(ch * 16, 16)] = pos0
            pb1[pl.ds(ch * 16, 16)] = pos1
        pltpu.sync_copy(pb0, pos_hbm.at[pl.ds(base, TPW)])
        pltpu.sync_copy(pb1, pos_hbm.at[pl.ds(T + base, TPW)])

    return functools.partial(
        pl.kernel, body,
        out_type=[
            jax.ShapeDtypeStruct((cap, H), jnp.float32),
            jax.ShapeDtypeStruct((cap, 128), jnp.float32),
            jax.ShapeDtypeStruct((2 * T,), jnp.int32),
            jax.ShapeDtypeStruct((2, 16), jnp.int32),
        ],
        mesh=mesh,
        scratch_types=[
            pltpu.VMEM((TPW, 128), jnp.float32),
            pltpu.VMEM((16, H), jnp.float32),
            pltpu.VMEM((16, 128), jnp.float32),
            pltpu.VMEM((TPW,), jnp.int32),
            pltpu.VMEM((TPW,), jnp.int32),
            pltpu.VMEM((16,), jnp.float32),
            pltpu.VMEM((2, 16), jnp.int32),
        ],
        compiler_params=pltpu.CompilerParams(needs_layout_passes=False),
    )()


def _make_combine_sc(T, cap):
    NW = 32
    TPW = T // NW
    NCH = TPW // 16
    mesh = plsc.VectorSubcoreMesh(core_axis_name="c", subcore_axis_name="s")

    def body(rs_hbm, shared_hbm, pos_hbm, out_hbm, r0, r1, sh, idx0, idx1, s0, s1):
        cid = lax.axis_index("c")
        sid = lax.axis_index("s")
        wid = sid * 2 + cid
        base = wid * TPW
        for ch in range(NCH):
            t0 = base + ch * 16
            pltpu.sync_copy(pos_hbm.at[pl.ds(t0, 16)], idx0)
            pltpu.sync_copy(pos_hbm.at[pl.ds(T + t0, 16)], idx1)
            c0 = pltpu.async_copy(rs_hbm.at[idx0[...]], r0, s0)
            c1 = pltpu.async_copy(rs_hbm.at[idx1[...]], r1, s1)
            pltpu.sync_copy(shared_hbm.at[pl.ds(t0, 16)], sh)
            c0.wait()
            c1.wait()
            for j in range(16):
                def addrow(k, _, j=j):
                    for u in range(8):
                        sl = pl.ds(k * 128 + u * 16, 16)
                        sh[j, sl] = sh[j, sl] + r0[j, sl] + r1[j, sl]
                    return 0
                lax.fori_loop(0, H // 128, addrow, 0)
            pltpu.sync_copy(sh, out_hbm.at[pl.ds(t0, 16)])

    return functools.partial(
        pl.kernel, body,
        out_type=jax.ShapeDtypeStruct((T, H), jnp.float32),
        mesh=mesh,
        scratch_types=[
            pltpu.VMEM((16, H), jnp.float32),
            pltpu.VMEM((16, H), jnp.float32),
            pltpu.VMEM((16, H), jnp.float32),
            pltpu.VMEM((16,), jnp.int32),
            pltpu.VMEM((16,), jnp.int32),
            pltpu.SemaphoreType.DMA,
            pltpu.SemaphoreType.DMA,
        ],
        compiler_params=pltpu.CompilerParams(needs_layout_passes=False),
    )()


def kernel(hidden_states, Wr, br, e_bias, Wg, bg, Wu, bu, Wd, bd, Wgs, bgs, Wus, bus, Wds, bds):
    orig_shape = hidden_states.shape
    x = hidden_states.reshape(-1, H).astype(jnp.float32)
    T = x.shape[0]
    cap = 2 * T + N_GROUP * BT
    cap_tiles = cap // BT
    maxt = T // BT

    wrp = jnp.pad(Wr, ((0, 0), (0, 128 - E)))
    brp = jnp.pad(br, (0, 128 - E)).reshape(1, 128)
    scores = pl.pallas_call(
        _router_body,
        grid=(T // 512,),
        in_specs=[
            pl.BlockSpec((512, H), lambda i: (i, 0)),
            pl.BlockSpec((H, 128), lambda i: (0, 0)),
            pl.BlockSpec((1, 128), lambda i: (0, 0)),
        ],
        out_specs=pl.BlockSpec((512, 128), lambda i: (i, 0)),
        out_shape=jax.ShapeDtypeStruct((T, 128), jnp.float32),
    )(x, wrp, brp)

    xg, wg, pos, meta = _make_route_sc(T, cap)(scores, x, e_bias)

    grid_spec = pltpu.PrefetchScalarGridSpec(
        num_scalar_prefetch=1,
        grid=(E, maxt),
        in_specs=[
            pl.BlockSpec(
                (BT, H),
                lambda e, i, m: (m[0, e // GSZ] + jnp.minimum(i, jnp.maximum(m[1, e // GSZ] - 1, 0)), 0)),
            pl.BlockSpec((cap, 128), lambda e, i, m: (0, 0)),
            pl.BlockSpec((1, H, INTER), lambda e, i, m: (e, 0, 0)),
            pl.BlockSpec((1, H, INTER), lambda e, i, m: (e, 0, 0)),
            pl.BlockSpec((1, INTER, H), lambda e, i, m: (e, 0, 0)),
            pl.BlockSpec((1, 1, INTER), lambda e, i, m: (e, 0, 0)),
            pl.BlockSpec((1, 1, INTER), lambda e, i, m: (e, 0, 0)),
            pl.BlockSpec((1, 1, H), lambda e, i, m: (e, 0, 0)),
        ],
        out_specs=pl.BlockSpec((cap, H), lambda e, i, m: (0, 0)),
    )
    routed_sorted = pl.pallas_call(
        functools.partial(_moe_body, cap_tiles=cap_tiles),
        grid_spec=grid_spec,
        out_shape=jax.ShapeDtypeStruct((cap, H), jnp.float32),
    )(meta, xg, wg, Wg, Wu, Wd,
      bg.reshape(E, 1, INTER), bu.reshape(E, 1, INTER), bd.reshape(E, 1, H))

    shared = pl.pallas_call(
        _shared_body,
        grid=(T // BT,),
        in_specs=[
            pl.BlockSpec((BT, H), lambda i: (i, 0)),
            pl.BlockSpec((H, SI), lambda i: (0, 0)),
            pl.BlockSpec((1, SI), lambda i: (0, 0)),
            pl.BlockSpec((H, SI), lambda i: (0, 0)),
            pl.BlockSpec((1, SI), lambda i: (0, 0)),
            pl.BlockSpec((SI, H), lambda i: (0, 0)),
            pl.BlockSpec((1, H), lambda i: (0, 0)),
        ],
        out_specs=pl.BlockSpec((BT, H), lambda i: (i, 0)),
        out_shape=jax.ShapeDtypeStruct((T, H), jnp.float32),
    )(x, Wgs, bgs.reshape(1, SI), Wus, bus.reshape(1, SI), Wds, bds.reshape(1, H))

    out = _make_combine_sc(T, cap)(routed_sorted, shared, pos)
    return out.reshape(orig_shape)


# speedup vs baseline: 1.2132x; 1.0815x over previous
import functools

import jax
import jax.numpy as jnp
from jax import lax
from jax.experimental import pallas as pl
from jax.experimental.pallas import tpu as pltpu
from jax.experimental.pallas import tpu_sc as plsc

H = 1024
E = 16
N_GROUP = 4
GSZ = E // N_GROUP
TOPK_GROUP = 2
INTER = 512
SI = 1024
SCALE = 2.5

BT = 256
LOG_BT = 8


def _router_body(x_ref, wr_ref, br_ref, o_ref):
    logits = jnp.dot(x_ref[...], wr_ref[...], preferred_element_type=jnp.float32)
    o_ref[...] = jax.nn.sigmoid(logits + br_ref[...])


def _moe_body(meta_ref, xg_ref, wg_ref, wgk_ref, wuk_ref, wdk_ref,
              bg_ref, bu_ref, bd_ref, out_ref, *, cap_tiles):
    e = pl.program_id(0)
    i = pl.program_id(1)
    g = e // GSZ

    @pl.when((e == 0) & (i == 0))
    def _zero():
        out_ref[...] = jnp.zeros_like(out_ref)

    nt = meta_ref[1, g]

    @pl.when(i < nt)
    def _compute():
        xt = xg_ref[...]
        gg = jnp.dot(xt, wgk_ref[0], preferred_element_type=jnp.float32) + bg_ref[0]
        uu = jnp.dot(xt, wuk_ref[0], preferred_element_type=jnp.float32) + bu_ref[0]
        hh = gg * jax.nn.sigmoid(gg) * uu
        dd = jnp.dot(hh, wdk_ref[0], preferred_element_type=jnp.float32) + bd_ref[0]
        row0 = (meta_ref[0, g] + i) * BT
        w_all = wg_ref[pl.ds(row0, BT), :]
        ids = lax.broadcasted_iota(jnp.int32, (1, 128), 1)
        wcol = jnp.sum(w_all * jnp.where(ids == e, 1.0, 0.0), axis=1, keepdims=True)
        out_ref[pl.ds(row0, BT), :] += dd * wcol


def _shared_body(x_ref, wgs_ref, bgs_ref, wus_ref, bus_ref, wds_ref, bds_ref, o_ref):
    xt = x_ref[...]
    a = jnp.dot(xt, wgs_ref[...], preferred_element_type=jnp.float32) + bgs_ref[...]
    b = jnp.dot(xt, wus_ref[...], preferred_element_type=jnp.float32) + bus_ref[...]
    h = a * jax.nn.sigmoid(a) * b
    o_ref[...] = jnp.dot(h, wds_ref[...], preferred_element_type=jnp.float32) + bds_ref[...]


def _group_select(cols, ebs):
    sfc = [cols[e] + ebs[e] for e in range(E)]
    gsv = []
    for g in range(N_GROUP):
        c0, c1, c2, c3 = sfc[4 * g:4 * g + 4]
        lo01 = jnp.minimum(c0, c1)
        hi01 = jnp.maximum(c0, c1)
        lo23 = jnp.minimum(c2, c3)
        hi23 = jnp.maximum(c2, c3)
        mn = jnp.minimum(lo01, lo23)
        mn2 = jnp.minimum(jnp.maximum(lo01, lo23), jnp.minimum(hi01, hi23))
        gsv.append(c0 + c1 + c2 + c3 - mn - mn2)
    sel = []
    for g in range(N_GROUP):
        beats = jnp.zeros((16,), jnp.int32)
        for j in range(N_GROUP):
            if j == g:
                continue
            if j < g:
                m = gsv[j] >= gsv[g]
            else:
                m = gsv[j] > gsv[g]
            beats = beats + m.astype(jnp.int32)
        sel.append((beats < TOPK_GROUP).astype(jnp.int32))
    return sel


def _make_route_sc(T, cap):
    NW = 32
    TPW = T // NW
    NCH = TPW // 16
    mesh = plsc.VectorSubcoreMesh(core_axis_name="c", subcore_axis_name="s")

    def body(scores_hbm, x_hbm, ebias_hbm, xg_hbm, wg_hbm, pos_hbm, meta_hbm,
             sc_scores, sc_big, xbuf, cmbbuf, pb0, pb1, ebv, metas):
        wid = lax.axis_index("s") * 2 + lax.axis_index("c")
        base = wid * TPW
        iota16 = lax.iota(jnp.int32, 16)
        onehot = [(iota16 == g).astype(jnp.int32) for g in range(N_GROUP)]
        pltpu.sync_copy(ebias_hbm, ebv)
        ebvec = ebv[...]
        ebs = [jnp.take_along_axis(ebvec, jnp.full((16,), e, jnp.int32), axis=0)
               for e in range(E)]

        def get_cols(ch, ref=None):
            ref = sc_scores if ref is None else ref
            rows = ch * 16 + iota16
            return [plsc.load_gather(
                ref, [rows, jnp.full((16,), e, jnp.int32)])
                for e in range(E)]

        def blk_body(blk, carry):
            pre, tot = carry
            pltpu.sync_copy(scores_hbm.at[pl.ds(blk * (4 * TPW), 4 * TPW)], sc_big)
            for sb in range(4):
                cntb = jnp.zeros((16,), jnp.int32)
                for ch in range(NCH):
                    sel = _group_select(get_cols(sb * NCH + ch, sc_big), ebs)
                    for g in range(N_GROUP):
                        cntb = cntb + jnp.sum(sel[g]) * onehot[g]
                tot = tot + cntb
                pre = pre + jnp.where(blk * 4 + sb < wid, cntb, 0)
            return pre, tot

        pre, tot = lax.fori_loop(
            0, NW // 4, blk_body,
            (jnp.zeros((16,), jnp.int32), jnp.zeros((16,), jnp.int32)))
        pltpu.sync_copy(scores_hbm.at[pl.ds(base, TPW)], sc_scores)
        aligned = ((tot + BT - 1) >> LOG_BT) << LOG_BT
        aligned = jnp.where(iota16 < N_GROUP, aligned, 0)
        cums = plsc.cumsum(aligned)
        off = cums - aligned
        basev = off + pre

        @pl.when(wid == 0)
        def _meta():
            metas[0, :] = off >> LOG_BT
            metas[1, :] = aligned >> LOG_BT
            pltpu.sync_copy(metas, meta_hbm)

        z16 = jnp.zeros((16,), jnp.float32)
        for r in range(16):
            for k in range(8):
                cmbbuf[r, pl.ds(k * 16, 16)] = z16

        rb = basev
        for ch in range(NCH):
            tok0 = base + ch * 16
            pltpu.sync_copy(x_hbm.at[pl.ds(tok0, 16)], xbuf)
            cols = get_cols(ch)
            sel = _group_select(cols, ebs)
            wts = [cols[e] * sel[e // GSZ].astype(jnp.float32) for e in range(E)]
            denom = functools.reduce(lambda a, b: a + b, wts) + 1e-20
            inv = SCALE / denom
            for e in range(E):
                plsc.store_scatter(
                    cmbbuf, [iota16, jnp.full((16,), e, jnp.int32)], wts[e] * inv)
            before = jnp.zeros((16,), jnp.int32)
            pos0 = jnp.zeros((16,), jnp.int32)
            pos1 = jnp.zeros((16,), jnp.int32)
            for g in range(N_GROUP):
                sg = sel[g]
                excl = plsc.cumsum(sg) - sg
                pg = jnp.take_along_axis(
                    rb, jnp.full((16,), g, jnp.int32), axis=0) + excl
                pos0 = pos0 + jnp.where((sg > 0) & (before == 0), pg, 0)
                pos1 = pos1 + jnp.where((sg > 0) & (before == 1), pg, 0)
                before = before + sg
                rb = rb + jnp.sum(sg) * onehot[g]
            pltpu.sync_copy(xbuf, xg_hbm.at[pos0])
            pltpu.sync_copy(xbuf, xg_hbm.at[pos1])
            pltpu.sync_copy(cmbbuf, wg_hbm.at[pos0])
            pltpu.sync_copy(cmbbuf, wg_hbm.at[pos1])
            pb0[pl.ds(ch * 16, 16)] = pos0
            pb1[pl.ds(ch * 16, 16)] = pos1
        pltpu.sync_copy(pb0, pos_hbm.at[pl.ds(base, TPW)])
        pltpu.sync_copy(pb1, pos_hbm.at[pl.ds(T + base, TPW)])

    return functools.partial(
        pl.kernel, body,
        out_type=[
            jax.ShapeDtypeStruct((cap, H), jnp.float32),
            jax.ShapeDtypeStruct((cap, 128), jnp.float32),
            jax.ShapeDtypeStruct((2 * T,), jnp.int32),
            jax.ShapeDtypeStruct((2, 16), jnp.int32),
        ],
        mesh=mesh,
        scratch_types=[
            pltpu.VMEM((TPW, 128), jnp.float32),
            pltpu.VMEM((4 * TPW, 128), jnp.float32),
            pltpu.VMEM((16, H), jnp.float32),
            pltpu.VMEM((16, 128), jnp.float32),
            pltpu.VMEM((TPW,), jnp.int32),
            pltpu.VMEM((TPW,), jnp.int32),
            pltpu.VMEM((16,), jnp.float32),
            pltpu.VMEM((2, 16), jnp.int32),
        ],
        compiler_params=pltpu.CompilerParams(needs_layout_passes=False),
    )()


def _make_combine_sc(T, cap):
    NW = 32
    TPW = T // NW
    NCH = TPW // 16
    mesh = plsc.VectorSubcoreMesh(core_axis_name="c", subcore_axis_name="s")

    def body(rs_hbm, shared_hbm, pos_hbm, out_hbm, r0, r1, sh, idx0, idx1, s0, s1):
        cid = lax.axis_index("c")
        sid = lax.axis_index("s")
        wid = sid * 2 + cid
        base = wid * TPW
        for ch in range(NCH):
            t0 = base + ch * 16
            pltpu.sync_copy(pos_hbm.at[pl.ds(t0, 16)], idx0)
            pltpu.sync_copy(pos_hbm.at[pl.ds(T + t0, 16)], idx1)
            c0 = pltpu.async_copy(rs_hbm.at[idx0[...]], r0, s0)
            c1 = pltpu.async_copy(rs_hbm.at[idx1[...]], r1, s1)
            pltpu.sync_copy(shared_hbm.at[pl.ds(t0, 16)], sh)
            c0.wait()
            c1.wait()
            for j in range(16):
                def addrow(k, _, j=j):
                    for u in range(8):
                        sl = pl.ds(k * 128 + u * 16, 16)
                        sh[j, sl] = sh[j, sl] + r0[j, sl] + r1[j, sl]
                    return 0
                lax.fori_loop(0, H // 128, addrow, 0)
            pltpu.sync_copy(sh, out_hbm.at[pl.ds(t0, 16)])

    return functools.partial(
        pl.kernel, body,
        out_type=jax.ShapeDtypeStruct((T, H), jnp.float32),
        mesh=mesh,
        scratch_types=[
            pltpu.VMEM((16, H), jnp.float32),
            pltpu.VMEM((16, H), jnp.float32),
            pltpu.VMEM((16, H), jnp.float32),
            pltpu.VMEM((16,), jnp.int32),
            pltpu.VMEM((16,), jnp.int32),
            pltpu.SemaphoreType.DMA,
            pltpu.SemaphoreType.DMA,
        ],
        compiler_params=pltpu.CompilerParams(needs_layout_passes=False),
    )()


def kernel(hidden_states, Wr, br, e_bias, Wg, bg, Wu, bu, Wd, bd, Wgs, bgs, Wus, bus, Wds, bds):
    orig_shape = hidden_states.shape
    x = hidden_states.reshape(-1, H).astype(jnp.float32)
    T = x.shape[0]
    cap = 2 * T + N_GROUP * BT
    cap_tiles = cap // BT
    maxt = T // BT

    wrp = jnp.pad(Wr, ((0, 0), (0, 128 - E)))
    brp = jnp.pad(br, (0, 128 - E)).reshape(1, 128)
    scores = pl.pallas_call(
        _router_body,
        grid=(T // 512,),
        in_specs=[
            pl.BlockSpec((512, H), lambda i: (i, 0)),
            pl.BlockSpec((H, 128), lambda i: (0, 0)),
            pl.BlockSpec((1, 128), lambda i: (0, 0)),
        ],
        out_specs=pl.BlockSpec((512, 128), lambda i: (i, 0)),
        out_shape=jax.ShapeDtypeStruct((T, 128), jnp.float32),
    )(x, wrp, brp)

    xg, wg, pos, meta = _make_route_sc(T, cap)(scores, x, e_bias)

    grid_spec = pltpu.PrefetchScalarGridSpec(
        num_scalar_prefetch=1,
        grid=(E, maxt),
        in_specs=[
            pl.BlockSpec(
                (BT, H),
                lambda e, i, m: (m[0, e // GSZ] + jnp.minimum(i, jnp.maximum(m[1, e // GSZ] - 1, 0)), 0)),
            pl.BlockSpec((cap, 128), lambda e, i, m: (0, 0)),
            pl.BlockSpec((1, H, INTER), lambda e, i, m: (e, 0, 0)),
            pl.BlockSpec((1, H, INTER), lambda e, i, m: (e, 0, 0)),
            pl.BlockSpec((1, INTER, H), lambda e, i, m: (e, 0, 0)),
            pl.BlockSpec((1, 1, INTER), lambda e, i, m: (e, 0, 0)),
            pl.BlockSpec((1, 1, INTER), lambda e, i, m: (e, 0, 0)),
            pl.BlockSpec((1, 1, H), lambda e, i, m: (e, 0, 0)),
        ],
        out_specs=pl.BlockSpec((cap, H), lambda e, i, m: (0, 0)),
    )
    routed_sorted = pl.pallas_call(
        functools.partial(_moe_body, cap_tiles=cap_tiles),
        grid_spec=grid_spec,
        out_shape=jax.ShapeDtypeStruct((cap, H), jnp.float32),
    )(meta, xg, wg, Wg, Wu, Wd,
      bg.reshape(E, 1, INTER), bu.reshape(E, 1, INTER), bd.reshape(E, 1, H))

    shared = pl.pallas_call(
        _shared_body,
        grid=(T // BT,),
        in_specs=[
            pl.BlockSpec((BT, H), lambda i: (i, 0)),
            pl.BlockSpec((H, SI), lambda i: (0, 0)),
            pl.BlockSpec((1, SI), lambda i: (0, 0)),
            pl.BlockSpec((H, SI), lambda i: (0, 0)),
            pl.BlockSpec((1, SI), lambda i: (0, 0)),
            pl.BlockSpec((SI, H), lambda i: (0, 0)),
            pl.BlockSpec((1, H), lambda i: (0, 0)),
        ],
        out_specs=pl.BlockSpec((BT, H), lambda i: (i, 0)),
        out_shape=jax.ShapeDtypeStruct((T, H), jnp.float32),
    )(x, Wgs, bgs.reshape(1, SI), Wus, bus.reshape(1, SI), Wds, bds.reshape(1, H))

    out = _make_combine_sc(T, cap)(routed_sorted, shared, pos)
    return out.reshape(orig_shape)
